# baseline (speedup 1.0000x reference)
import functools

import jax
import jax.numpy as jnp
from jax import lax
from jax.experimental import pallas as pl
from jax.experimental.pallas import tpu as pltpu
from jax.experimental.pallas import tpu_sc as plsc

_L = 16
_IDX_CHUNK = 128


def _embed_body(n_chunks, ids_hbm, table_hbm, out_hbm, idx_v, rows_v, sem):
    wid = lax.axis_index("s") * 2 + lax.axis_index("c")
    b_per_w = n_chunks * _IDX_CHUNK
    base = wid * b_per_w

    pltpu.sync_copy(ids_hbm.at[wid], idx_v)

    for r in range(n_chunks):
        row = idx_v.at[r]
        for c in range(_IDX_CHUNK // _L):
            v = row[pl.ds(c * _L, _L)]
            v = jnp.where((v == 1) | (v == 2), 0, v)
            row[pl.ds(c * _L, _L)] = v

    copies = []
    for r in range(n_chunks):
        copies.append(
            pltpu.async_copy(
                table_hbm.at[idx_v.at[r]],
                rows_v.at[pl.ds(r * _IDX_CHUNK, _IDX_CHUNK)],
                sem,
            )
        )
    for cp in copies:
        cp.wait()

    pltpu.sync_copy(rows_v, out_hbm.at[pl.ds(base, b_per_w)])


def kernel(dataset_ids, table):
    batch = dataset_ids.shape[0]
    n_tables, embed = table.shape
    info = plsc.get_sparse_core_info()
    nw = info.num_cores * info.num_subcores

    b_per_w = batch // nw
    n_chunks = b_per_w // _IDX_CHUNK
    assert b_per_w * nw == batch and n_chunks * _IDX_CHUNK == b_per_w

    mesh = plsc.VectorSubcoreMesh(core_axis_name="c", subcore_axis_name="s")
    run = functools.partial(
        pl.kernel,
        mesh=mesh,
        out_type=jax.ShapeDtypeStruct((batch, embed), jnp.float32),
        scratch_types=[
            pltpu.VMEM((n_chunks, _IDX_CHUNK), jnp.int32),
            pltpu.VMEM((b_per_w, embed), jnp.float32),
            pltpu.SemaphoreType.DMA,
        ],
    )(functools.partial(_embed_body, n_chunks))

    ids3 = dataset_ids.astype(jnp.int32).reshape(nw, n_chunks, _IDX_CHUNK)
    return run(ids3, table)


# speedup vs baseline: 6.6269x; 6.6269x over previous
import functools

import jax
import jax.numpy as jnp
from jax import lax
from jax.experimental import pallas as pl
from jax.experimental.pallas import tpu as pltpu
from jax.experimental.pallas import tpu_sc as plsc

_L = 16


def _embed_body(b_per_w, embed, ids_hbm, table_hbm, out_hbm, idx_v, table_v,
                out_v):
    wid = lax.axis_index("s") * 2 + lax.axis_index("c")
    n_groups = b_per_w // _L
    d_chunks = embed // _L

    pltpu.sync_copy(ids_hbm.at[wid], idx_v)
    pltpu.sync_copy(table_hbm, table_v)

    def prep(g, carry):
        v = idx_v[pl.ds(g * _L, _L)]
        v = jnp.where((v == 1) | (v == 2), 0, v)
        idx_v[pl.ds(g * _L, _L)] = v
        return carry

    lax.fori_loop(0, n_groups, prep, 0)

    def group(g, carry):
        v = idx_v[pl.ds(g * _L, _L)]
        for j in range(_L):
            row = g * _L + j
            rid = v[j]
            for c in range(d_chunks):
                out_v[pl.ds(row * embed + c * _L, _L)] = (
                    table_v[rid, pl.ds(c * _L, _L)])
        return carry

    lax.fori_loop(0, n_groups, group, 0)

    pltpu.sync_copy(out_v, out_hbm.at[pl.ds(wid * b_per_w * embed,
                                            b_per_w * embed)])


def kernel(dataset_ids, table):
    batch = dataset_ids.shape[0]
    n_tables, embed = table.shape
    info = plsc.get_sparse_core_info()
    nw = info.num_cores * info.num_subcores

    b_per_w = batch // nw
    assert b_per_w * nw == batch and b_per_w % _L == 0 and embed % _L == 0

    mesh = plsc.VectorSubcoreMesh(core_axis_name="c", subcore_axis_name="s")
    run = functools.partial(
        pl.kernel,
        mesh=mesh,
        out_type=jax.ShapeDtypeStruct((batch * embed,), jnp.float32),
        scratch_types=[
            pltpu.VMEM((b_per_w,), jnp.int32),
            pltpu.VMEM((n_tables, embed), jnp.float32),
            pltpu.VMEM((b_per_w * embed,), jnp.float32),
        ],
    )(functools.partial(_embed_body, b_per_w, embed))

    ids2 = dataset_ids.astype(jnp.int32).reshape(nw, b_per_w)
    out = run(ids2, table)
    return out.reshape(batch, embed)


# speedup vs baseline: 9.1575x; 1.3819x over previous
import functools

import jax
import jax.numpy as jnp
from jax import lax
from jax.experimental import pallas as pl
from jax.experimental.pallas import tpu as pltpu
from jax.experimental.pallas import tpu_sc as plsc

_L = 16


def _embed_body(b_per_w, embed, ids_hbm, table_hbm, out_hbm, idx_v, table_v,
                out_v):
    wid = lax.axis_index("s") * 2 + lax.axis_index("c")
    n_groups = b_per_w // _L
    d_chunks = embed // _L

    pltpu.sync_copy(ids_hbm.at[wid], idx_v)
    pltpu.sync_copy(table_hbm, table_v)

    def prep(g, carry):
        v = idx_v[pl.ds(g * _L, _L)]
        v = jnp.where((v == 1) | (v == 2), 0, v)
        idx_v[pl.ds(g * _L, _L)] = v
        return carry

    lax.fori_loop(0, n_groups, prep, 0)

    def group(g, carry):
        v = idx_v[pl.ds(g * _L, _L)]
        for j in range(_L):
            row = g * _L + j
            rid = v[j]
            chunks = [table_v[rid, pl.ds(c * _L, _L)] for c in range(d_chunks)]
            for c in range(d_chunks):
                out_v[pl.ds(row * embed + c * _L, _L)] = chunks[c]
        return carry

    lax.fori_loop(0, n_groups, group, 0)

    pltpu.sync_copy(out_v, out_hbm.at[pl.ds(wid * b_per_w * embed,
                                            b_per_w * embed)])


def kernel(dataset_ids, table):
    batch = dataset_ids.shape[0]
    n_tables, embed = table.shape
    info = plsc.get_sparse_core_info()
    nw = info.num_cores * info.num_subcores

    b_per_w = batch // nw
    assert b_per_w * nw == batch and b_per_w % _L == 0 and embed % _L == 0

    mesh = plsc.VectorSubcoreMesh(core_axis_name="c", subcore_axis_name="s")
    run = functools.partial(
        pl.kernel,
        mesh=mesh,
        out_type=jax.ShapeDtypeStruct((batch * embed,), jnp.float32),
        scratch_types=[
            pltpu.VMEM((b_per_w,), jnp.int32),
            pltpu.VMEM((n_tables, embed), jnp.float32),
            pltpu.VMEM((b_per_w * embed,), jnp.float32),
        ],
    )(functools.partial(_embed_body, b_per_w, embed))

    ids2 = dataset_ids.astype(jnp.int32).reshape(nw, b_per_w)
    out = run(ids2, table)
    return out.reshape(batch, embed)
